# baseline (speedup 1.0000x reference)
import functools

import jax
import jax.numpy as jnp
from jax import lax
from jax.experimental import pallas as pl
from jax.experimental.pallas import tpu as pltpu
from jax.experimental.pallas import tpu_sc as plsc


def _build_gather(B, V, D, NC, NS):
    NW = NC * NS
    BPW = B // NW
    CHUNK = 8
    NCH = BPW // CHUNK
    NBUF = 3

    mesh = plsc.VectorSubcoreMesh(core_axis_name="c", subcore_axis_name="s")

    @functools.partial(
        pl.kernel,
        mesh=mesh,
        out_type=jax.ShapeDtypeStruct((B, D), jnp.float32),
        scratch_types=[
            pltpu.VMEM((BPW,), jnp.int32),
            pltpu.VMEM((CHUNK, D), jnp.float32),
            pltpu.VMEM((CHUNK, D), jnp.float32),
            pltpu.VMEM((CHUNK, D), jnp.float32),
            pltpu.SemaphoreType.DMA,
            pltpu.SemaphoreType.DMA,
            pltpu.SemaphoreType.DMA,
            pltpu.SemaphoreType.DMA,
            pltpu.SemaphoreType.DMA,
            pltpu.SemaphoreType.DMA,
        ],
    )
    def k(table_hbm, ids_hbm, out_hbm, idx_v,
          buf0, buf1, buf2, gs0, gs1, gs2, ss0, ss1, ss2):
        wid = lax.axis_index("s") * NC + lax.axis_index("c")
        base = wid * BPW
        pltpu.sync_copy(ids_hbm.at[pl.ds(base, BPW)], idx_v)

        bufs = (buf0, buf1, buf2)
        gsems = (gs0, gs1, gs2)
        ssems = (ss0, ss1, ss2)

        def g_start(j, b):
            pltpu.async_copy(
                table_hbm.at[idx_v.at[pl.ds(j * CHUNK, CHUNK)]], bufs[b], gsems[b]
            )

        def g_wait(j, b):
            pltpu.make_async_copy(
                table_hbm.at[idx_v.at[pl.ds(j * CHUNK, CHUNK)]], bufs[b], gsems[b]
            ).wait()

        def s_start(j, b):
            pltpu.async_copy(bufs[b], out_hbm.at[pl.ds(base + j * CHUNK, CHUNK)], ssems[b])

        def s_wait(j, b):
            pltpu.make_async_copy(
                bufs[b], out_hbm.at[pl.ds(base + j * CHUNK, CHUNK)], ssems[b]
            ).wait()

        g_start(0, 0)
        g_wait(0, 0); s_start(0, 0); g_start(1, 1)
        g_wait(1, 1); s_start(1, 1); g_start(2, 2)
        g_wait(2, 2); s_start(2, 2); s_wait(0, 0); g_start(3, 0)

        def body(j, b):
            g_wait(j, b)
            s_start(j, b)
            s_wait(j - 2, (b + 1) % NBUF)
            g_start(j + 1, (b + 1) % NBUF)

        F = (NCH - 4) // NBUF

        def group(gi, carry):
            j0 = NBUF + gi * NBUF
            for u in range(NBUF):
                body(j0 + u, u)
            return carry

        lax.fori_loop(0, F, group, 0)

        for j in range(NBUF + NBUF * F, NCH - 1):
            body(j, j % NBUF)

        jl = NCH - 1
        bl = jl % NBUF
        g_wait(jl, bl)
        s_wait(jl - 2, (jl - 2) % NBUF)
        s_wait(jl - 1, (jl - 1) % NBUF)
        pltpu.sync_copy(bufs[bl], out_hbm.at[pl.ds(base + jl * CHUNK, CHUNK)])

    return k


def _tc_gather(table, ids, D):
    B_tc = ids.shape[0]
    table3 = table.reshape(table.shape[0], 1, D)

    def body(ids_ref, row_ref, out_ref):
        out_ref[...] = row_ref[...]

    out = pl.pallas_call(
        body,
        grid_spec=pltpu.PrefetchScalarGridSpec(
            num_scalar_prefetch=1,
            grid=(B_tc,),
            in_specs=[
                pl.BlockSpec((1, 1, D), lambda i, ids_ref: (ids_ref[i], 0, 0)),
            ],
            out_specs=pl.BlockSpec((1, 1, D), lambda i, ids_ref: (i, 0, 0)),
        ),
        out_shape=jax.ShapeDtypeStruct((B_tc, 1, D), jnp.float32),
    )(ids, table3)
    return out.reshape(B_tc, D)


def kernel(input_ids, embed_weight):
    V, D = embed_weight.shape
    B = input_ids.size
    info = plsc.get_sparse_core_info()
    ids_flat = input_ids.reshape(-1).astype(jnp.int32)
    B_sc = (B * 11 // 16) // 256 * 256
    gather = _build_gather(B_sc, V, D, info.num_cores, info.num_subcores)
    sc_out = gather(embed_weight, ids_flat[:B_sc])
    tc_out = _tc_gather(embed_weight, ids_flat[B_sc:], D)
    out = jnp.concatenate([sc_out, tc_out], axis=0)
    return out.reshape(*input_ids.shape, D)


# speedup vs baseline: 5.9868x; 5.9868x over previous
import functools

import jax
import jax.numpy as jnp
from jax import lax
from jax.experimental import pallas as pl
from jax.experimental.pallas import tpu as pltpu
from jax.experimental.pallas import tpu_sc as plsc


def _build_gather(B, V, D, NC, NS):
    NW = NC * NS
    BPW = B // NW
    CHUNK = 8
    NCH = BPW // CHUNK
    NBUF = 3

    mesh = plsc.VectorSubcoreMesh(core_axis_name="c", subcore_axis_name="s")

    @functools.partial(
        pl.kernel,
        mesh=mesh,
        out_type=jax.ShapeDtypeStruct((B, D), jnp.float32),
        scratch_types=[
            pltpu.VMEM((BPW,), jnp.int32),
            pltpu.VMEM((CHUNK, D), jnp.float32),
            pltpu.VMEM((CHUNK, D), jnp.float32),
            pltpu.VMEM((CHUNK, D), jnp.float32),
            pltpu.SemaphoreType.DMA,
            pltpu.SemaphoreType.DMA,
            pltpu.SemaphoreType.DMA,
            pltpu.SemaphoreType.DMA,
            pltpu.SemaphoreType.DMA,
            pltpu.SemaphoreType.DMA,
        ],
    )
    def k(table_hbm, ids_hbm, out_hbm, idx_v,
          buf0, buf1, buf2, gs0, gs1, gs2, ss0, ss1, ss2):
        wid = lax.axis_index("s") * NC + lax.axis_index("c")
        base = wid * BPW
        pltpu.sync_copy(ids_hbm.at[pl.ds(base, BPW)], idx_v)

        bufs = (buf0, buf1, buf2)
        gsems = (gs0, gs1, gs2)
        ssems = (ss0, ss1, ss2)

        def g_start(j, b):
            pltpu.async_copy(
                table_hbm.at[idx_v.at[pl.ds(j * CHUNK, CHUNK)]], bufs[b], gsems[b]
            )

        def g_wait(j, b):
            pltpu.make_async_copy(
                table_hbm.at[idx_v.at[pl.ds(j * CHUNK, CHUNK)]], bufs[b], gsems[b]
            ).wait()

        def s_start(j, b):
            pltpu.async_copy(bufs[b], out_hbm.at[pl.ds(base + j * CHUNK, CHUNK)], ssems[b])

        def s_wait(j, b):
            pltpu.make_async_copy(
                bufs[b], out_hbm.at[pl.ds(base + j * CHUNK, CHUNK)], ssems[b]
            ).wait()

        g_start(0, 0)
        g_wait(0, 0); s_start(0, 0); g_start(1, 1)
        g_wait(1, 1); s_start(1, 1); g_start(2, 2)
        g_wait(2, 2); s_start(2, 2); s_wait(0, 0); g_start(3, 0)

        def body(j, b):
            g_wait(j, b)
            s_start(j, b)
            s_wait(j - 2, (b + 1) % NBUF)
            g_start(j + 1, (b + 1) % NBUF)

        F = (NCH - 4) // NBUF

        def group(gi, carry):
            j0 = NBUF + gi * NBUF
            for u in range(NBUF):
                body(j0 + u, u)
            return carry

        lax.fori_loop(0, F, group, 0)

        for j in range(NBUF + NBUF * F, NCH - 1):
            body(j, j % NBUF)

        jl = NCH - 1
        bl = jl % NBUF
        g_wait(jl, bl)
        s_wait(jl - 2, (jl - 2) % NBUF)
        s_wait(jl - 1, (jl - 1) % NBUF)
        pltpu.sync_copy(bufs[bl], out_hbm.at[pl.ds(base + jl * CHUNK, CHUNK)])

    return k


def _tc_gather(table, ids, D):
    B_tc = ids.shape[0]
    K = 8
    NG = 4
    G = B_tc // K

    def body(ids_ref, table_ref, out_ref, b0, b1, b2, b3,
             is0, is1, is2, is3, os0, os1, os2, os3):
        bufs = (b0, b1, b2, b3)
        isems = (is0, is1, is2, is3)
        osems = (os0, os1, os2, os3)

        def in_start(g, b):
            for u in range(K):
                idx = ids_ref[g * K + u]
                pltpu.make_async_copy(
                    table_ref.at[pl.ds(idx, 1)], bufs[b].at[pl.ds(u, 1)], isems[b]
                ).start()

        def in_wait(b):
            pltpu.make_async_copy(
                table_ref.at[pl.ds(0, K)], bufs[b], isems[b]
            ).wait()

        def out_start(g, b):
            pltpu.make_async_copy(
                bufs[b], out_ref.at[pl.ds(g * K, K)], osems[b]
            ).start()

        def out_wait(g, b):
            pltpu.make_async_copy(
                bufs[b], out_ref.at[pl.ds(g * K, K)], osems[b]
            ).wait()

        in_start(0, 0)
        in_start(1, 1)
        in_wait(0); out_start(0, 0); in_start(2, 2)
        in_wait(1); out_start(1, 1); in_start(3, 3)

        def piter(g, b):
            in_wait(b)
            out_start(g, b)
            out_wait(g - 2, (b + 2) % NG)
            in_start(g + 2, (b + 2) % NG)

        F = (G - 6) // NG

        def ring(r, carry):
            g0 = 2 + r * NG
            for u in range(NG):
                piter(g0 + u, (2 + u) % NG)
            return carry

        lax.fori_loop(0, F, ring, 0)
        for g in range(2 + NG * F, G - 2):
            piter(g, g % NG)
        for g in range(G - 2, G):
            b = g % NG
            in_wait(b)
            out_start(g, b)
            out_wait(g - 2, (g + 2) % NG)
        for g in range(G - 2, G):
            out_wait(g, g % NG)

    out = pl.pallas_call(
        body,
        in_specs=[
            pl.BlockSpec(memory_space=pltpu.SMEM),
            pl.BlockSpec(memory_space=pl.ANY),
        ],
        out_specs=pl.BlockSpec(memory_space=pl.ANY),
        out_shape=jax.ShapeDtypeStruct((B_tc, D), jnp.float32),
        scratch_shapes=(
            [pltpu.VMEM((K, D), jnp.float32) for _ in range(NG)]
            + [pltpu.SemaphoreType.DMA] * (2 * NG)
        ),
    )(ids, table)
    return out


def kernel(input_ids, embed_weight):
    V, D = embed_weight.shape
    B = input_ids.size
    info = plsc.get_sparse_core_info()
    ids_flat = input_ids.reshape(-1).astype(jnp.int32)
    B_sc = (B * 25 // 32) // 256 * 256
    gather = _build_gather(B_sc, V, D, info.num_cores, info.num_subcores)
    sc_out = gather(embed_weight, ids_flat[:B_sc])
    tc_out = _tc_gather(embed_weight, ids_flat[B_sc:], D)
    out = jnp.concatenate([sc_out, tc_out], axis=0)
    return out.reshape(*input_ids.shape, D)


# speedup vs baseline: 7.2522x; 1.2114x over previous
import functools

import jax
import jax.numpy as jnp
from jax import lax
from jax.experimental import pallas as pl
from jax.experimental.pallas import tpu as pltpu
from jax.experimental.pallas import tpu_sc as plsc


def _build_gather(B, V, D, NC, NS):
    NW = NC * NS
    BPW = B // NW
    CHUNK = 8
    NCH = BPW // CHUNK
    NBUF = 3

    mesh = plsc.VectorSubcoreMesh(core_axis_name="c", subcore_axis_name="s")

    @functools.partial(
        pl.kernel,
        mesh=mesh,
        out_type=jax.ShapeDtypeStruct((B, D), jnp.float32),
        scratch_types=[
            pltpu.VMEM((BPW,), jnp.int32),
            pltpu.VMEM((CHUNK, D), jnp.float32),
            pltpu.VMEM((CHUNK, D), jnp.float32),
            pltpu.VMEM((CHUNK, D), jnp.float32),
            pltpu.SemaphoreType.DMA,
            pltpu.SemaphoreType.DMA,
            pltpu.SemaphoreType.DMA,
            pltpu.SemaphoreType.DMA,
            pltpu.SemaphoreType.DMA,
            pltpu.SemaphoreType.DMA,
        ],
    )
    def k(table_hbm, ids_hbm, out_hbm, idx_v,
          buf0, buf1, buf2, gs0, gs1, gs2, ss0, ss1, ss2):
        wid = lax.axis_index("s") * NC + lax.axis_index("c")
        base = wid * BPW
        pltpu.sync_copy(ids_hbm.at[pl.ds(base, BPW)], idx_v)

        bufs = (buf0, buf1, buf2)
        gsems = (gs0, gs1, gs2)
        ssems = (ss0, ss1, ss2)

        def g_start(j, b):
            pltpu.async_copy(
                table_hbm.at[idx_v.at[pl.ds(j * CHUNK, CHUNK)]], bufs[b], gsems[b]
            )

        def g_wait(j, b):
            pltpu.make_async_copy(
                table_hbm.at[idx_v.at[pl.ds(j * CHUNK, CHUNK)]], bufs[b], gsems[b]
            ).wait()

        def s_start(j, b):
            pltpu.async_copy(bufs[b], out_hbm.at[pl.ds(base + j * CHUNK, CHUNK)], ssems[b])

        def s_wait(j, b):
            pltpu.make_async_copy(
                bufs[b], out_hbm.at[pl.ds(base + j * CHUNK, CHUNK)], ssems[b]
            ).wait()

        g_start(0, 0)
        g_wait(0, 0); s_start(0, 0); g_start(1, 1)
        g_wait(1, 1); s_start(1, 1); g_start(2, 2)
        g_wait(2, 2); s_start(2, 2); s_wait(0, 0); g_start(3, 0)

        def body(j, b):
            g_wait(j, b)
            s_start(j, b)
            s_wait(j - 2, (b + 1) % NBUF)
            g_start(j + 1, (b + 1) % NBUF)

        F = (NCH - 4) // NBUF

        def group(gi, carry):
            j0 = NBUF + gi * NBUF
            for u in range(NBUF):
                body(j0 + u, u)
            return carry

        lax.fori_loop(0, F, group, 0)

        for j in range(NBUF + NBUF * F, NCH - 1):
            body(j, j % NBUF)

        jl = NCH - 1
        bl = jl % NBUF
        g_wait(jl, bl)
        s_wait(jl - 2, (jl - 2) % NBUF)
        s_wait(jl - 1, (jl - 1) % NBUF)
        pltpu.sync_copy(bufs[bl], out_hbm.at[pl.ds(base + jl * CHUNK, CHUNK)])

    return k


def _tc_gather(table, ids, D):
    B_tc = ids.shape[0]
    K = 8
    NG = 4
    G = B_tc // K

    def body(ids_ref, table_ref, out_ref, b0, b1, b2, b3,
             is0, is1, is2, is3, os0, os1, os2, os3):
        bufs = (b0, b1, b2, b3)
        isems = (is0, is1, is2, is3)
        osems = (os0, os1, os2, os3)

        def in_start(g, b):
            for u in range(K):
                idx = ids_ref[g * K + u]
                pltpu.make_async_copy(
                    table_ref.at[pl.ds(idx, 1)], bufs[b].at[pl.ds(u, 1)], isems[b]
                ).start()

        def in_wait(b):
            pltpu.make_async_copy(
                table_ref.at[pl.ds(0, K)], bufs[b], isems[b]
            ).wait()

        def out_start(g, b):
            pltpu.make_async_copy(
                bufs[b], out_ref.at[pl.ds(g * K, K)], osems[b]
            ).start()

        def out_wait(g, b):
            pltpu.make_async_copy(
                bufs[b], out_ref.at[pl.ds(g * K, K)], osems[b]
            ).wait()

        in_start(0, 0)
        in_start(1, 1)
        in_wait(0); out_start(0, 0); in_start(2, 2)
        in_wait(1); out_start(1, 1); in_start(3, 3)

        def piter(g, b):
            in_wait(b)
            out_start(g, b)
            out_wait(g - 2, (b + 2) % NG)
            in_start(g + 2, (b + 2) % NG)

        F = (G - 6) // NG

        def ring(r, carry):
            g0 = 2 + r * NG
            for u in range(NG):
                piter(g0 + u, (2 + u) % NG)
            return carry

        lax.fori_loop(0, F, ring, 0)
        for g in range(2 + NG * F, G - 2):
            piter(g, g % NG)
        for g in range(G - 2, G):
            b = g % NG
            in_wait(b)
            out_start(g, b)
            out_wait(g - 2, (g + 2) % NG)
        for g in range(G - 2, G):
            out_wait(g, g % NG)

    out = pl.pallas_call(
        body,
        in_specs=[
            pl.BlockSpec(memory_space=pltpu.SMEM),
            pl.BlockSpec(memory_space=pl.ANY),
        ],
        out_specs=pl.BlockSpec(memory_space=pl.ANY),
        out_shape=jax.ShapeDtypeStruct((B_tc, D), jnp.float32),
        scratch_shapes=(
            [pltpu.VMEM((K, D), jnp.float32) for _ in range(NG)]
            + [pltpu.SemaphoreType.DMA] * (2 * NG)
        ),
    )(ids, table)
    return out


def kernel(input_ids, embed_weight):
    V, D = embed_weight.shape
    B = input_ids.size
    info = plsc.get_sparse_core_info()
    ids_flat = input_ids.reshape(-1).astype(jnp.int32)
    B_sc = (B * 15 // 16) // 256 * 256
    gather = _build_gather(B_sc, V, D, info.num_cores, info.num_subcores)
    sc_out = gather(embed_weight, ids_flat[:B_sc])
    tc_out = _tc_gather(embed_weight, ids_flat[B_sc:], D)
    out = jnp.concatenate([sc_out, tc_out], axis=0)
    return out.reshape(*input_ids.shape, D)


# speedup vs baseline: 13.2702x; 1.8298x over previous
import functools

import jax
import jax.numpy as jnp
from jax import lax
from jax.experimental import pallas as pl
from jax.experimental.pallas import tpu as pltpu
from jax.experimental.pallas import tpu_sc as plsc


def _build_gather(B, V, D, NC, NS):
    NW = NC * NS
    BPW = B // NW
    CHUNK = 8
    NCH = BPW // CHUNK
    NBUF = 3

    mesh = plsc.VectorSubcoreMesh(core_axis_name="c", subcore_axis_name="s")

    @functools.partial(
        pl.kernel,
        mesh=mesh,
        out_type=jax.ShapeDtypeStruct((B, D), jnp.float32),
        scratch_types=[
            pltpu.VMEM((BPW,), jnp.int32),
            pltpu.VMEM((CHUNK, D), jnp.float32),
            pltpu.VMEM((CHUNK, D), jnp.float32),
            pltpu.VMEM((CHUNK, D), jnp.float32),
            pltpu.SemaphoreType.DMA,
            pltpu.SemaphoreType.DMA,
            pltpu.SemaphoreType.DMA,
            pltpu.SemaphoreType.DMA,
            pltpu.SemaphoreType.DMA,
            pltpu.SemaphoreType.DMA,
        ],
    )
    def k(table_hbm, ids_hbm, out_hbm, idx_v,
          buf0, buf1, buf2, gs0, gs1, gs2, ss0, ss1, ss2):
        wid = lax.axis_index("s") * NC + lax.axis_index("c")
        base = wid * BPW
        pltpu.sync_copy(ids_hbm.at[pl.ds(base, BPW)], idx_v)

        bufs = (buf0, buf1, buf2)
        gsems = (gs0, gs1, gs2)
        ssems = (ss0, ss1, ss2)

        def g_start(j, b):
            pltpu.async_copy(
                table_hbm.at[idx_v.at[pl.ds(j * CHUNK, CHUNK)]], bufs[b], gsems[b]
            )

        def g_wait(j, b):
            pltpu.make_async_copy(
                table_hbm.at[idx_v.at[pl.ds(j * CHUNK, CHUNK)]], bufs[b], gsems[b]
            ).wait()

        def s_start(j, b):
            pltpu.async_copy(bufs[b], out_hbm.at[pl.ds(base + j * CHUNK, CHUNK)], ssems[b])

        def s_wait(j, b):
            pltpu.make_async_copy(
                bufs[b], out_hbm.at[pl.ds(base + j * CHUNK, CHUNK)], ssems[b]
            ).wait()

        g_start(0, 0)
        g_wait(0, 0); s_start(0, 0); g_start(1, 1)
        g_wait(1, 1); s_start(1, 1); g_start(2, 2)
        g_wait(2, 2); s_start(2, 2); s_wait(0, 0); g_start(3, 0)

        def body(j, b):
            g_wait(j, b)
            s_start(j, b)
            s_wait(j - 2, (b + 1) % NBUF)
            g_start(j + 1, (b + 1) % NBUF)

        F = (NCH - 4) // NBUF

        def group(gi, carry):
            j0 = NBUF + gi * NBUF
            for u in range(NBUF):
                body(j0 + u, u)
            return carry

        lax.fori_loop(0, F, group, 0)

        for j in range(NBUF + NBUF * F, NCH - 1):
            body(j, j % NBUF)

        jl = NCH - 1
        bl = jl % NBUF
        g_wait(jl, bl)
        s_wait(jl - 2, (jl - 2) % NBUF)
        s_wait(jl - 1, (jl - 1) % NBUF)
        pltpu.sync_copy(bufs[bl], out_hbm.at[pl.ds(base + jl * CHUNK, CHUNK)])

    return k


def _tc_gather(table, ids, D):
    B_tc = ids.shape[0]
    K = 8
    NG = 4
    G = B_tc // K

    def body(ids_ref, table_ref, out_ref, b0, b1, b2, b3,
             is0, is1, is2, is3, os0, os1, os2, os3):
        bufs = (b0, b1, b2, b3)
        isems = (is0, is1, is2, is3)
        osems = (os0, os1, os2, os3)

        def in_start(g, b):
            for u in range(K):
                idx = ids_ref[g * K + u]
                pltpu.make_async_copy(
                    table_ref.at[pl.ds(idx, 1)], bufs[b].at[pl.ds(u, 1)], isems[b]
                ).start()

        def in_wait(b):
            pltpu.make_async_copy(
                table_ref.at[pl.ds(0, K)], bufs[b], isems[b]
            ).wait()

        def out_start(g, b):
            pltpu.make_async_copy(
                bufs[b], out_ref.at[pl.ds(g * K, K)], osems[b]
            ).start()

        def out_wait(g, b):
            pltpu.make_async_copy(
                bufs[b], out_ref.at[pl.ds(g * K, K)], osems[b]
            ).wait()

        in_start(0, 0)
        in_start(1, 1)
        in_wait(0); out_start(0, 0); in_start(2, 2)
        in_wait(1); out_start(1, 1); in_start(3, 3)

        def piter(g, b):
            in_wait(b)
            out_start(g, b)
            out_wait(g - 2, (b + 2) % NG)
            in_start(g + 2, (b + 2) % NG)

        F = (G - 6) // NG

        def ring(r, carry):
            g0 = 2 + r * NG
            for u in range(NG):
                piter(g0 + u, (2 + u) % NG)
            return carry

        lax.fori_loop(0, F, ring, 0)
        for g in range(2 + NG * F, G - 2):
            piter(g, g % NG)
        for g in range(G - 2, G):
            b = g % NG
            in_wait(b)
            out_start(g, b)
            out_wait(g - 2, (g + 2) % NG)
        for g in range(G - 2, G):
            out_wait(g, g % NG)

    out = pl.pallas_call(
        body,
        in_specs=[
            pl.BlockSpec(memory_space=pltpu.SMEM),
            pl.BlockSpec(memory_space=pl.ANY),
        ],
        out_specs=pl.BlockSpec(memory_space=pl.ANY),
        out_shape=jax.ShapeDtypeStruct((B_tc, D), jnp.float32),
        scratch_shapes=(
            [pltpu.VMEM((K, D), jnp.float32) for _ in range(NG)]
            + [pltpu.SemaphoreType.DMA] * (2 * NG)
        ),
    )(ids, table)
    return out


def kernel(input_ids, embed_weight):
    V, D = embed_weight.shape
    B = input_ids.size
    info = plsc.get_sparse_core_info()
    ids_flat = input_ids.reshape(-1).astype(jnp.int32)
    gather = _build_gather(B, V, D, info.num_cores, info.num_subcores)
    out = gather(embed_weight, ids_flat)
    return out.reshape(*input_ids.shape, D)


# speedup vs baseline: 13.3140x; 1.0033x over previous
import functools

import jax
import jax.numpy as jnp
from jax import lax
from jax.experimental import pallas as pl
from jax.experimental.pallas import tpu as pltpu
from jax.experimental.pallas import tpu_sc as plsc


def _build_gather(B, V, D, NC, NS):
    NW = NC * NS
    BPW = B // NW
    CHUNK = 8
    NCH = BPW // CHUNK
    NBUF = 3

    mesh = plsc.VectorSubcoreMesh(core_axis_name="c", subcore_axis_name="s")

    @functools.partial(
        pl.kernel,
        mesh=mesh,
        out_type=jax.ShapeDtypeStruct((B, D), jnp.float32),
        scratch_types=[
            pltpu.VMEM((BPW,), jnp.int32),
            pltpu.VMEM((CHUNK, D), jnp.float32),
            pltpu.VMEM((CHUNK, D), jnp.float32),
            pltpu.VMEM((CHUNK, D), jnp.float32),
            pltpu.SemaphoreType.DMA,
            pltpu.SemaphoreType.DMA,
            pltpu.SemaphoreType.DMA,
            pltpu.SemaphoreType.DMA,
            pltpu.SemaphoreType.DMA,
            pltpu.SemaphoreType.DMA,
        ],
    )
    def k(table_hbm, ids_hbm, out_hbm, idx_v,
          buf0, buf1, buf2, gs0, gs1, gs2, ss0, ss1, ss2):
        wid = lax.axis_index("s") * NC + lax.axis_index("c")
        base = wid * BPW
        pltpu.sync_copy(ids_hbm.at[pl.ds(base, BPW)], idx_v)

        bufs = (buf0, buf1, buf2)
        gsems = (gs0, gs1, gs2)
        ssems = (ss0, ss1, ss2)

        def g_start(j, b):
            pltpu.async_copy(
                table_hbm.at[idx_v.at[pl.ds(j * CHUNK, CHUNK)]], bufs[b], gsems[b]
            )

        def g_wait(j, b):
            pltpu.make_async_copy(
                table_hbm.at[idx_v.at[pl.ds(j * CHUNK, CHUNK)]], bufs[b], gsems[b]
            ).wait()

        def s_start(j, b):
            pltpu.async_copy(bufs[b], out_hbm.at[pl.ds(base + j * CHUNK, CHUNK)], ssems[b])

        def s_wait(j, b):
            pltpu.make_async_copy(
                bufs[b], out_hbm.at[pl.ds(base + j * CHUNK, CHUNK)], ssems[b]
            ).wait()

        g_start(0, 0)
        g_start(1, 1)
        g_wait(0, 0); s_start(0, 0); g_start(2, 2)
        g_wait(1, 1); s_start(1, 1); s_wait(0, 0); g_start(3, 0)

        def body(j, b):
            g_wait(j, b)
            s_start(j, b)
            s_wait(j - 1, (b + 2) % NBUF)
            g_start(j + 2, (b + 2) % NBUF)

        F = (NCH - 4) // NBUF

        def group(gi, carry):
            j0 = 2 + gi * NBUF
            for u in range(NBUF):
                body(j0 + u, (2 + u) % NBUF)
            return carry

        lax.fori_loop(0, F, group, 0)

        for j in range(2 + NBUF * F, NCH - 2):
            body(j, j % NBUF)

        for j in range(NCH - 2, NCH):
            b = j % NBUF
            g_wait(j, b)
            s_start(j, b)
            s_wait(j - 1, (j - 1) % NBUF)
        s_wait(NCH - 1, (NCH - 1) % NBUF)

    return k


def _tc_gather(table, ids, D):
    B_tc = ids.shape[0]
    K = 8
    NG = 4
    G = B_tc // K

    def body(ids_ref, table_ref, out_ref, b0, b1, b2, b3,
             is0, is1, is2, is3, os0, os1, os2, os3):
        bufs = (b0, b1, b2, b3)
        isems = (is0, is1, is2, is3)
        osems = (os0, os1, os2, os3)

        def in_start(g, b):
            for u in range(K):
                idx = ids_ref[g * K + u]
                pltpu.make_async_copy(
                    table_ref.at[pl.ds(idx, 1)], bufs[b].at[pl.ds(u, 1)], isems[b]
                ).start()

        def in_wait(b):
            pltpu.make_async_copy(
                table_ref.at[pl.ds(0, K)], bufs[b], isems[b]
            ).wait()

        def out_start(g, b):
            pltpu.make_async_copy(
                bufs[b], out_ref.at[pl.ds(g * K, K)], osems[b]
            ).start()

        def out_wait(g, b):
            pltpu.make_async_copy(
                bufs[b], out_ref.at[pl.ds(g * K, K)], osems[b]
            ).wait()

        in_start(0, 0)
        in_start(1, 1)
        in_wait(0); out_start(0, 0); in_start(2, 2)
        in_wait(1); out_start(1, 1); in_start(3, 3)

        def piter(g, b):
            in_wait(b)
            out_start(g, b)
            out_wait(g - 2, (b + 2) % NG)
            in_start(g + 2, (b + 2) % NG)

        F = (G - 6) // NG

        def ring(r, carry):
            g0 = 2 + r * NG
            for u in range(NG):
                piter(g0 + u, (2 + u) % NG)
            return carry

        lax.fori_loop(0, F, ring, 0)
        for g in range(2 + NG * F, G - 2):
            piter(g, g % NG)
        for g in range(G - 2, G):
            b = g % NG
            in_wait(b)
            out_start(g, b)
            out_wait(g - 2, (g + 2) % NG)
        for g in range(G - 2, G):
            out_wait(g, g % NG)

    out = pl.pallas_call(
        body,
        in_specs=[
            pl.BlockSpec(memory_space=pltpu.SMEM),
            pl.BlockSpec(memory_space=pl.ANY),
        ],
        out_specs=pl.BlockSpec(memory_space=pl.ANY),
        out_shape=jax.ShapeDtypeStruct((B_tc, D), jnp.float32),
        scratch_shapes=(
            [pltpu.VMEM((K, D), jnp.float32) for _ in range(NG)]
            + [pltpu.SemaphoreType.DMA] * (2 * NG)
        ),
    )(ids, table)
    return out


def kernel(input_ids, embed_weight):
    V, D = embed_weight.shape
    B = input_ids.size
    info = plsc.get_sparse_core_info()
    ids_flat = input_ids.reshape(-1).astype(jnp.int32)
    gather = _build_gather(B, V, D, info.num_cores, info.num_subcores)
    out = gather(embed_weight, ids_flat)
    return out.reshape(*input_ids.shape, D)
